# baseline (speedup 1.0000x reference)
import functools

import jax
import jax.numpy as jnp
from jax import lax
from jax.experimental import pallas as pl
from jax.experimental.pallas import tpu as pltpu
from jax.experimental.pallas import tpu_sc as plsc

N = 3125
NP = 3328
B = 8
TASKS = 2 * B
HALF = NP // 2
NCH = NP // 16


def _prep_body(logit_ref, lab_ref, bbox_ref,
               iou_o, prob_o, e1_o, f1_o, e2_o, f2_o, p_o):
    bb = bbox_ref[0]
    x1 = bb[0:1, :]
    y1 = bb[1:2, :]
    x2 = bb[2:3, :]
    y2 = bb[3:4, :]
    tx1 = bb[4:5, :]
    ty1 = bb[5:6, :]
    tx2 = bb[6:7, :]
    ty2 = bb[7:8, :]
    ww = jnp.clip(jnp.minimum(tx2, x2) - jnp.maximum(tx1, x1), 0.0, None)
    hh = jnp.clip(jnp.minimum(ty2, y2) - jnp.maximum(ty1, y1), 0.0, None)
    area = (x2 - x1) * (y2 - y1)
    ta = (tx2 - tx1) * (ty2 - ty1)
    inter = ww * hh
    iou = inter / (area + ta - inter)

    pos = lab_ref[0] > 0.0
    prob = jnp.exp(logit_ref[0])
    pf = jnp.sum(jnp.where(pos, 1.0, 0.0))
    pmin = jnp.min(jnp.where(pos, prob, jnp.inf))
    pmax = jnp.max(jnp.where(pos, prob, -jnp.inf))
    s1 = 0.5 * (pmin + pmax)

    iou_o[0] = iou
    prob_o[0] = prob
    e1_o[0] = jnp.where(pos, jnp.exp(prob - s1), 0.0)
    f1_o[0] = jnp.where(pos, jnp.exp(s1 - prob), 0.0)
    e2_o[0] = jnp.where(pos, jnp.exp(iou - 0.5), 0.0)
    f2_o[0] = jnp.where(pos, jnp.exp(0.5 - iou), 0.0)
    p_o[0] = jnp.broadcast_to(pf, (1, 128))


def _sc_pair_body(keys_hbm, es_hbm, fs_hbm, out_hbm, kv, ev, fv, accv):
    c = lax.axis_index("c")
    s = lax.axis_index("s")
    wid = s * 2 + c
    task = wid // 2
    half = wid % 2
    off = half * HALF

    pltpu.sync_copy(keys_hbm.at[task], kv)
    pltpu.sync_copy(es_hbm.at[task], ev)
    pltpu.sync_copy(fs_hbm.at[task, pl.ds(off, HALF)], fv)

    iota = lax.iota(jnp.int32, 16)

    def u_loop(ub, acc):
        u0 = off + ub * 16
        ku16 = kv[pl.ds(u0, 16)]
        fu16 = fv[pl.ds(ub * 16, 16)]
        for j in range(16):
            ku = ku16[j]
            fu = fu16[j]
            u = u0 + j

            def v_loop(cj, a):
                v0 = cj * 16
                kk = kv[pl.ds(v0, 16)]
                ee = ev[pl.ds(v0, 16)]
                iv = v0 + iota
                cond = (kk < ku) | ((kk == ku) & (iv > u))
                return a + jnp.where(cond, ee, 0.0)

            av = lax.fori_loop(0, NCH, v_loop,
                               jnp.zeros((16,), jnp.float32))
            acc = acc + fu * av
        return acc

    acc = lax.fori_loop(0, HALF // 16, u_loop, jnp.zeros((16,), jnp.float32))
    accv[...] = acc
    pltpu.sync_copy(accv, out_hbm.at[wid])


@jax.jit
def kernel(cls, label_cls, pred_bboxes, label_target):
    logit = cls.reshape(B, N, 2)[:, :, 1]
    logit = jnp.pad(logit, ((0, 0), (0, NP - N))).reshape(B, 1, NP)
    lab = jnp.pad(label_cls.reshape(B, N).astype(jnp.float32),
                  ((0, 0), (0, NP - N))).reshape(B, 1, NP)
    tgt = jnp.broadcast_to(label_target[:, :, None], (B, 4, N))
    bbox = jnp.pad(jnp.concatenate([pred_bboxes, tgt], axis=1),
                   ((0, 0), (0, 0), (0, NP - N)))

    row = pl.BlockSpec((1, 1, NP), lambda b: (b, 0, 0))
    iou, prob, e1, f1, e2, f2, pcount = pl.pallas_call(
        _prep_body,
        grid=(B,),
        in_specs=[
            row, row,
            pl.BlockSpec((1, 8, NP), lambda b: (b, 0, 0)),
        ],
        out_specs=[row, row, row, row, row, row,
                   pl.BlockSpec((1, 1, 128), lambda b: (b, 0, 0))],
        out_shape=[jax.ShapeDtypeStruct((B, 1, NP), jnp.float32)] * 6
        + [jax.ShapeDtypeStruct((B, 1, 128), jnp.float32)],
    )(logit, lab, bbox)

    keys = jnp.stack([iou[:, 0, :], prob[:, 0, :]], axis=1).reshape(TASKS, NP)
    es = jnp.stack([e1[:, 0, :], e2[:, 0, :]], axis=1).reshape(TASKS, NP)
    fs = jnp.stack([f1[:, 0, :], f2[:, 0, :]], axis=1).reshape(TASKS, NP)

    sc_pair = functools.partial(
        pl.kernel,
        out_type=jax.ShapeDtypeStruct((2 * TASKS, 16), jnp.float32),
        mesh=plsc.VectorSubcoreMesh(core_axis_name="c", subcore_axis_name="s"),
        scratch_types=[
            pltpu.VMEM((NP,), jnp.float32),
            pltpu.VMEM((NP,), jnp.float32),
            pltpu.VMEM((HALF,), jnp.float32),
            pltpu.VMEM((16,), jnp.float32),
        ],
    )(_sc_pair_body)
    partials = sc_pair(keys, es, fs)

    sums = jnp.sum(partials.reshape(B, 2, 2 * 16), axis=2)
    p = pcount[:, 0, 0]
    cnt = p * (p - 1.0) * 0.5
    loss1 = sums[:, 0] / cnt
    loss2 = sums[:, 1] / cnt
    valid = (p > 1.0) & ~jnp.isnan(loss1) & ~jnp.isnan(loss2)
    l1 = jnp.where(valid, loss1, 0.0)
    l2 = jnp.where(valid, loss2, 0.0)
    nvalid = jnp.sum(valid.astype(jnp.float32))
    final1 = jnp.where(nvalid > 0, jnp.sum(l1) / nvalid, 0.0)
    final2 = jnp.where(nvalid > 0, jnp.sum(l2) / nvalid, 0.0)
    return (final1, final2)


# speedup vs baseline: 8.2166x; 8.2166x over previous
import functools

import jax
import jax.numpy as jnp
from jax import lax
from jax.experimental import pallas as pl
from jax.experimental.pallas import tpu as pltpu
from jax.experimental.pallas import tpu_sc as plsc

N = 3125
NP = 3328
B = 8
TASKS = 2 * B
HALF = NP // 2
NCH = NP // 16


def _prep_body(logit_ref, lab_ref, bbox_ref,
               iou_o, prob_o, e1_o, f1_o, e2_o, f2_o, p_o):
    bb = bbox_ref[0]
    x1 = bb[0:1, :]
    y1 = bb[1:2, :]
    x2 = bb[2:3, :]
    y2 = bb[3:4, :]
    tx1 = bb[4:5, :]
    ty1 = bb[5:6, :]
    tx2 = bb[6:7, :]
    ty2 = bb[7:8, :]
    ww = jnp.clip(jnp.minimum(tx2, x2) - jnp.maximum(tx1, x1), 0.0, None)
    hh = jnp.clip(jnp.minimum(ty2, y2) - jnp.maximum(ty1, y1), 0.0, None)
    area = (x2 - x1) * (y2 - y1)
    ta = (tx2 - tx1) * (ty2 - ty1)
    inter = ww * hh
    iou = inter / (area + ta - inter)

    pos = lab_ref[0] > 0.0
    prob = jnp.exp(logit_ref[0])
    pf = jnp.sum(jnp.where(pos, 1.0, 0.0))
    pmin = jnp.min(jnp.where(pos, prob, jnp.inf))
    pmax = jnp.max(jnp.where(pos, prob, -jnp.inf))
    s1 = 0.5 * (pmin + pmax)

    iou_o[0] = iou
    prob_o[0] = prob
    e1_o[0] = jnp.where(pos, jnp.exp(prob - s1), 0.0)
    f1_o[0] = jnp.where(pos, jnp.exp(s1 - prob), 0.0)
    e2_o[0] = jnp.where(pos, jnp.exp(iou - 0.5), 0.0)
    f2_o[0] = jnp.where(pos, jnp.exp(0.5 - iou), 0.0)
    p_o[0] = jnp.broadcast_to(pf, (1, 128))


def _sc_pair_body(keys_hbm, es_hbm, fs_hbm, out_hbm,
                  kv, ev, fv, kc, ec, fc, ic, accv):
    c = lax.axis_index("c")
    s = lax.axis_index("s")
    wid = s * 2 + c
    task = wid // 2
    half = wid % 2

    pltpu.sync_copy(keys_hbm.at[task], kv)
    pltpu.sync_copy(es_hbm.at[task], ev)
    pltpu.sync_copy(fs_hbm.at[task], fv)

    iota = lax.iota(jnp.int32, 16)
    zero16 = jnp.zeros((16,), jnp.float32)

    def comp_loop(cj, cnt):
        v0 = cj * 16
        ee = ev[pl.ds(v0, 16)]
        m = ee != 0.0
        cs = jnp.where(m, 1, 0)
        for k in (1, 2, 4, 8):
            g = cs.at[jnp.maximum(iota - k, 0)].get(mode="promise_in_bounds")
            cs = cs + jnp.where(iota >= k, g, 0)
        idx = cnt + cs - 1
        plsc.store_scatter(kc, [idx], kv[pl.ds(v0, 16)], mask=m)
        plsc.store_scatter(ec, [idx], ee, mask=m)
        plsc.store_scatter(fc, [idx], fv[pl.ds(v0, 16)], mask=m)
        plsc.store_scatter(ic, [idx], v0 + iota, mask=m)
        return cnt + cs[15]

    pc = lax.fori_loop(0, NCH, comp_loop, jnp.int32(0))
    kc[pl.ds(pc, 16)] = zero16
    ec[pl.ds(pc, 16)] = zero16
    fc[pl.ds(pc, 16)] = zero16
    nb = (pc + 15) // 16
    ncv = nb
    b0 = jnp.where(half == 0, 0, nb // 2)
    b1 = jnp.where(half == 0, nb // 2, nb)

    def u_loop(ub, acc):
        u0 = ub * 16
        ku16 = kc[pl.ds(u0, 16)]
        fu16 = fc[pl.ds(u0, 16)]
        iu16 = ic[pl.ds(u0, 16)]
        ee_d = ec[pl.ds(u0, 16)]
        for g in range(4):
            ku = [ku16[4 * g + j] for j in range(4)]
            fu = [fu16[4 * g + j] for j in range(4)]
            iu = [iu16[4 * g + j] for j in range(4)]

            def v_lt(cj, a):
                v0 = cj * 16
                kk = kc[pl.ds(v0, 16)]
                ee = ec[pl.ds(v0, 16)]
                return tuple(a[j] + jnp.where(kk < ku[j], ee, 0.0)
                             for j in range(4))

            def v_le(cj, a):
                v0 = cj * 16
                kk = kc[pl.ds(v0, 16)]
                ee = ec[pl.ds(v0, 16)]
                return tuple(a[j] + jnp.where(kk <= ku[j], ee, 0.0)
                             for j in range(4))

            a4 = lax.fori_loop(0, ub, v_lt, (zero16,) * 4)
            a4 = lax.fori_loop(ub + 1, ncv, v_le, a4)
            for j in range(4):
                cond = (ku16 < ku[j]) | ((ku16 == ku[j]) & (iu16 > iu[j]))
                av = a4[j] + jnp.where(cond, ee_d, 0.0)
                acc = acc + fu[j] * av
        return acc

    acc = lax.fori_loop(b0, b1, u_loop, zero16)
    accv[...] = acc
    pltpu.sync_copy(accv, out_hbm.at[wid])


@jax.jit
def kernel(cls, label_cls, pred_bboxes, label_target):
    logit = cls.reshape(B, N, 2)[:, :, 1]
    logit = jnp.pad(logit, ((0, 0), (0, NP - N))).reshape(B, 1, NP)
    lab = jnp.pad(label_cls.reshape(B, N).astype(jnp.float32),
                  ((0, 0), (0, NP - N))).reshape(B, 1, NP)
    tgt = jnp.broadcast_to(label_target[:, :, None], (B, 4, N))
    bbox = jnp.pad(jnp.concatenate([pred_bboxes, tgt], axis=1),
                   ((0, 0), (0, 0), (0, NP - N)))

    row = pl.BlockSpec((1, 1, NP), lambda b: (b, 0, 0))
    iou, prob, e1, f1, e2, f2, pcount = pl.pallas_call(
        _prep_body,
        grid=(B,),
        in_specs=[
            row, row,
            pl.BlockSpec((1, 8, NP), lambda b: (b, 0, 0)),
        ],
        out_specs=[row, row, row, row, row, row,
                   pl.BlockSpec((1, 1, 128), lambda b: (b, 0, 0))],
        out_shape=[jax.ShapeDtypeStruct((B, 1, NP), jnp.float32)] * 6
        + [jax.ShapeDtypeStruct((B, 1, 128), jnp.float32)],
    )(logit, lab, bbox)

    keys = jnp.stack([iou[:, 0, :], prob[:, 0, :]], axis=1).reshape(TASKS, NP)
    es = jnp.stack([e1[:, 0, :], e2[:, 0, :]], axis=1).reshape(TASKS, NP)
    fs = jnp.stack([f1[:, 0, :], f2[:, 0, :]], axis=1).reshape(TASKS, NP)

    sc_pair = functools.partial(
        pl.kernel,
        out_type=jax.ShapeDtypeStruct((2 * TASKS, 16), jnp.float32),
        mesh=plsc.VectorSubcoreMesh(core_axis_name="c", subcore_axis_name="s"),
        compiler_params=pltpu.CompilerParams(needs_layout_passes=False),
        scratch_types=[
            pltpu.VMEM((NP,), jnp.float32),
            pltpu.VMEM((NP,), jnp.float32),
            pltpu.VMEM((NP,), jnp.float32),
            pltpu.VMEM((NP + 16,), jnp.float32),
            pltpu.VMEM((NP + 16,), jnp.float32),
            pltpu.VMEM((NP + 16,), jnp.float32),
            pltpu.VMEM((NP + 16,), jnp.int32),
            pltpu.VMEM((16,), jnp.float32),
        ],
    )(_sc_pair_body)
    partials = sc_pair(keys, es, fs)

    sums = jnp.sum(partials.reshape(B, 2, 2 * 16), axis=2)
    p = pcount[:, 0, 0]
    cnt = p * (p - 1.0) * 0.5
    loss1 = sums[:, 0] / cnt
    loss2 = sums[:, 1] / cnt
    valid = (p > 1.0) & ~jnp.isnan(loss1) & ~jnp.isnan(loss2)
    l1 = jnp.where(valid, loss1, 0.0)
    l2 = jnp.where(valid, loss2, 0.0)
    nvalid = jnp.sum(valid.astype(jnp.float32))
    final1 = jnp.where(nvalid > 0, jnp.sum(l1) / nvalid, 0.0)
    final2 = jnp.where(nvalid > 0, jnp.sum(l2) / nvalid, 0.0)
    return (final1, final2)


# speedup vs baseline: 10.8725x; 1.3232x over previous
import functools

import jax
import jax.numpy as jnp
from jax import lax
from jax.experimental import pallas as pl
from jax.experimental.pallas import tpu as pltpu
from jax.experimental.pallas import tpu_sc as plsc

N = 3125
NP = 3328
B = 8
TASKS = 2 * B
HALF = NP // 2
NCH = NP // 16


def _prep_body(logit_ref, lab_ref, bbox_ref,
               keys_o, es_o, fs_o, p_o):
    bb = bbox_ref[0]
    x1 = bb[0:1, :]
    y1 = bb[1:2, :]
    x2 = bb[2:3, :]
    y2 = bb[3:4, :]
    tx1 = bb[4:5, :]
    ty1 = bb[5:6, :]
    tx2 = bb[6:7, :]
    ty2 = bb[7:8, :]
    ww = jnp.clip(jnp.minimum(tx2, x2) - jnp.maximum(tx1, x1), 0.0, None)
    hh = jnp.clip(jnp.minimum(ty2, y2) - jnp.maximum(ty1, y1), 0.0, None)
    area = (x2 - x1) * (y2 - y1)
    ta = (tx2 - tx1) * (ty2 - ty1)
    inter = ww * hh
    iou = inter / (area + ta - inter)

    pos = lab_ref[0] > 0.0
    prob = jnp.exp(logit_ref[0])
    pf = jnp.sum(jnp.where(pos, 1.0, 0.0))
    pmin = jnp.min(jnp.where(pos, prob, jnp.inf))
    pmax = jnp.max(jnp.where(pos, prob, -jnp.inf))
    s1 = 0.5 * (pmin + pmax)

    keys_o[0, 0:1] = iou
    keys_o[0, 1:2] = prob
    es_o[0, 0:1] = jnp.where(pos, jnp.exp(prob - s1), 0.0)
    es_o[0, 1:2] = jnp.where(pos, jnp.exp(iou - 0.5), 0.0)
    fs_o[0, 0:1] = jnp.where(pos, jnp.exp(s1 - prob), 0.0)
    fs_o[0, 1:2] = jnp.where(pos, jnp.exp(0.5 - iou), 0.0)
    p_o[0] = jnp.broadcast_to(pf, (1, 128))


def _sc_pair_body(keys_hbm, es_hbm, fs_hbm, out_hbm,
                  kv, ev, fv, kc, ec, fc, ic, accv):
    c = lax.axis_index("c")
    s = lax.axis_index("s")
    wid = s * 2 + c
    task = wid // 2
    half = wid % 2

    pltpu.sync_copy(keys_hbm.at[task], kv)
    pltpu.sync_copy(es_hbm.at[task], ev)
    pltpu.sync_copy(fs_hbm.at[task], fv)

    iota = lax.iota(jnp.int32, 16)
    zero16 = jnp.zeros((16,), jnp.float32)

    def comp_loop(cj, cnt):
        v0 = cj * 16
        ee = ev[pl.ds(v0, 16)]
        m = ee != 0.0
        cs = jnp.where(m, 1, 0)
        for k in (1, 2, 4, 8):
            g = cs.at[jnp.maximum(iota - k, 0)].get(mode="promise_in_bounds")
            cs = cs + jnp.where(iota >= k, g, 0)
        idx = cnt + cs - 1
        plsc.store_scatter(kc, [idx], kv[pl.ds(v0, 16)], mask=m)
        plsc.store_scatter(ec, [idx], ee, mask=m)
        plsc.store_scatter(fc, [idx], fv[pl.ds(v0, 16)], mask=m)
        plsc.store_scatter(ic, [idx], v0 + iota, mask=m)
        return cnt + cs[15]

    pc = lax.fori_loop(0, NCH, comp_loop, jnp.int32(0))
    kc[pl.ds(pc, 16)] = zero16
    ec[pl.ds(pc, 16)] = zero16
    fc[pl.ds(pc, 16)] = zero16
    nb = (pc + 15) // 16
    ncv = nb
    b0 = jnp.where(half == 0, 0, nb // 2)
    b1 = jnp.where(half == 0, nb // 2, nb)

    def u_loop(ub, acc):
        u0 = ub * 16
        ku16 = kc[pl.ds(u0, 16)]
        fu16 = fc[pl.ds(u0, 16)]
        iu16 = ic[pl.ds(u0, 16)]
        ee_d = ec[pl.ds(u0, 16)]
        for g in range(4):
            ku = [ku16[4 * g + j] for j in range(4)]
            fu = [fu16[4 * g + j] for j in range(4)]
            iu = [iu16[4 * g + j] for j in range(4)]

            @plsc.parallel_loop(0, ub, unroll=4, carry=(zero16,) * 4)
            def v_lt(cj, a):
                v0 = cj * 16
                kk = kc[pl.ds(v0, 16)]
                ee = ec[pl.ds(v0, 16)]
                return tuple(a[j] + jnp.where(kk < ku[j], ee, 0.0)
                             for j in range(4))

            @plsc.parallel_loop(ub + 1, ncv, unroll=4, carry=v_lt)
            def v_le(cj, a):
                v0 = cj * 16
                kk = kc[pl.ds(v0, 16)]
                ee = ec[pl.ds(v0, 16)]
                return tuple(a[j] + jnp.where(kk <= ku[j], ee, 0.0)
                             for j in range(4))

            a4 = v_le
            for j in range(4):
                cond = (ku16 < ku[j]) | ((ku16 == ku[j]) & (iu16 > iu[j]))
                av = a4[j] + jnp.where(cond, ee_d, 0.0)
                acc = acc + fu[j] * av
        return acc

    acc = lax.fori_loop(b0, b1, u_loop, zero16)
    accv[...] = acc
    pltpu.sync_copy(accv, out_hbm.at[wid])


@jax.jit
def kernel(cls, label_cls, pred_bboxes, label_target):
    logit = cls.reshape(B, N, 2)[:, :, 1]
    logit = jnp.pad(logit, ((0, 0), (0, NP - N))).reshape(B, 1, NP)
    lab = jnp.pad(label_cls.reshape(B, N).astype(jnp.float32),
                  ((0, 0), (0, NP - N))).reshape(B, 1, NP)
    tgt = jnp.broadcast_to(label_target[:, :, None], (B, 4, N))
    bbox = jnp.pad(jnp.concatenate([pred_bboxes, tgt], axis=1),
                   ((0, 0), (0, 0), (0, NP - N)))

    row = pl.BlockSpec((1, 1, NP), lambda b: (b, 0, 0))
    pair_row = pl.BlockSpec((1, 2, NP), lambda b: (b, 0, 0))
    keys3, es3, fs3, pcount = pl.pallas_call(
        _prep_body,
        grid=(B,),
        in_specs=[
            row, row,
            pl.BlockSpec((1, 8, NP), lambda b: (b, 0, 0)),
        ],
        out_specs=[pair_row, pair_row, pair_row,
                   pl.BlockSpec((1, 1, 128), lambda b: (b, 0, 0))],
        out_shape=[jax.ShapeDtypeStruct((B, 2, NP), jnp.float32)] * 3
        + [jax.ShapeDtypeStruct((B, 1, 128), jnp.float32)],
    )(logit, lab, bbox)

    keys = keys3.reshape(TASKS, NP)
    es = es3.reshape(TASKS, NP)
    fs = fs3.reshape(TASKS, NP)

    sc_pair = functools.partial(
        pl.kernel,
        out_type=jax.ShapeDtypeStruct((2 * TASKS, 16), jnp.float32),
        mesh=plsc.VectorSubcoreMesh(core_axis_name="c", subcore_axis_name="s"),
        compiler_params=pltpu.CompilerParams(needs_layout_passes=False),
        scratch_types=[
            pltpu.VMEM((NP,), jnp.float32),
            pltpu.VMEM((NP,), jnp.float32),
            pltpu.VMEM((NP,), jnp.float32),
            pltpu.VMEM((NP + 16,), jnp.float32),
            pltpu.VMEM((NP + 16,), jnp.float32),
            pltpu.VMEM((NP + 16,), jnp.float32),
            pltpu.VMEM((NP + 16,), jnp.int32),
            pltpu.VMEM((16,), jnp.float32),
        ],
    )(_sc_pair_body)
    partials = sc_pair(keys, es, fs)

    sums = jnp.sum(partials.reshape(B, 2, 2 * 16), axis=2)
    p = pcount[:, 0, 0]
    cnt = p * (p - 1.0) * 0.5
    loss1 = sums[:, 0] / cnt
    loss2 = sums[:, 1] / cnt
    valid = (p > 1.0) & ~jnp.isnan(loss1) & ~jnp.isnan(loss2)
    l1 = jnp.where(valid, loss1, 0.0)
    l2 = jnp.where(valid, loss2, 0.0)
    nvalid = jnp.sum(valid.astype(jnp.float32))
    final1 = jnp.where(nvalid > 0, jnp.sum(l1) / nvalid, 0.0)
    final2 = jnp.where(nvalid > 0, jnp.sum(l2) / nvalid, 0.0)
    return (final1, final2)


# speedup vs baseline: 11.1902x; 1.0292x over previous
import functools

import jax
import jax.numpy as jnp
from jax import lax
from jax.experimental import pallas as pl
from jax.experimental.pallas import tpu as pltpu
from jax.experimental.pallas import tpu_sc as plsc

N = 3125
NP = 3328
B = 8
TASKS = 2 * B
NCH = NP // 16


def _sc_body(logit_hbm, lab_hbm, bbox_hbm, tgt_hbm, out_hbm, pout_hbm,
             labv, probv, x1v, y1v, x2v, y2v, tgtv,
             kc, ec, fc, ic, accv, pcov):
    c = lax.axis_index("c")
    s = lax.axis_index("s")
    wid = s * 2 + c
    task = wid // 2
    half = wid % 2
    b = task // 2
    l0 = (task % 2) == 0

    pltpu.sync_copy(lab_hbm.at[b], labv)
    pltpu.sync_copy(logit_hbm.at[b], probv)
    pltpu.sync_copy(bbox_hbm.at[b, 0], x1v)
    pltpu.sync_copy(bbox_hbm.at[b, 1], y1v)
    pltpu.sync_copy(bbox_hbm.at[b, 2], x2v)
    pltpu.sync_copy(bbox_hbm.at[b, 3], y2v)
    pltpu.sync_copy(tgt_hbm.at[b], tgtv)
    t16 = tgtv[...]
    tx1 = t16[0]
    ty1 = t16[1]
    tx2 = t16[2]
    ty2 = t16[3]
    ta = (tx2 - tx1) * (ty2 - ty1)

    iota = lax.iota(jnp.int32, 16)
    zero16 = jnp.zeros((16,), jnp.float32)

    def prob_loop(cj, mm):
        v0 = cj * 16
        pos = labv[pl.ds(v0, 16)] > 0.0
        prob = jnp.exp(probv[pl.ds(v0, 16)])
        probv[pl.ds(v0, 16)] = prob
        mn = jnp.minimum(mm[0], jnp.where(pos, prob, jnp.inf))
        mx = jnp.maximum(mm[1], jnp.where(pos, prob, -jnp.inf))
        return (mn, mx)

    mn16, mx16 = lax.fori_loop(0, NCH, prob_loop,
                               (jnp.full((16,), jnp.inf, jnp.float32),
                                jnp.full((16,), -jnp.inf, jnp.float32)))
    s1 = 0.5 * (jnp.min(mn16) + jnp.max(mx16))
    sh = jnp.where(l0, s1, 0.5)

    def comp_loop(cj, cnt):
        v0 = cj * 16
        pos = labv[pl.ds(v0, 16)] > 0.0
        x1 = x1v[pl.ds(v0, 16)]
        y1 = y1v[pl.ds(v0, 16)]
        x2 = x2v[pl.ds(v0, 16)]
        y2 = y2v[pl.ds(v0, 16)]
        ww = jnp.maximum(jnp.minimum(tx2, x2) - jnp.maximum(tx1, x1), 0.0)
        hh = jnp.maximum(jnp.minimum(ty2, y2) - jnp.maximum(ty1, y1), 0.0)
        inter = ww * hh
        iou = inter / ((x2 - x1) * (y2 - y1) + ta - inter)
        prob = probv[pl.ds(v0, 16)]
        key = jnp.where(l0, iou, prob)
        val = jnp.where(l0, prob, iou)
        ee = jnp.where(pos, jnp.exp(val - sh), 0.0)
        ff = jnp.where(pos, jnp.exp(sh - val), 0.0)
        cs = jnp.where(pos, 1, 0)
        for k in (1, 2, 4, 8):
            g = cs.at[jnp.maximum(iota - k, 0)].get(mode="promise_in_bounds")
            cs = cs + jnp.where(iota >= k, g, 0)
        idx = cnt + cs - 1
        plsc.store_scatter(kc, [idx], key, mask=pos)
        plsc.store_scatter(ec, [idx], ee, mask=pos)
        plsc.store_scatter(fc, [idx], ff, mask=pos)
        plsc.store_scatter(ic, [idx], v0 + iota, mask=pos)
        return cnt + cs[15]

    pc = lax.fori_loop(0, NCH, comp_loop, jnp.int32(0))
    kc[pl.ds(pc, 16)] = zero16
    ec[pl.ds(pc, 16)] = zero16
    fc[pl.ds(pc, 16)] = zero16

    nb = (pc + 15) // 16
    ncv = nb
    b0 = jnp.where(half == 0, 0, nb // 2)
    b1 = jnp.where(half == 0, nb // 2, nb)

    def u_loop(ub, acc):
        u0 = ub * 16
        ku16 = kc[pl.ds(u0, 16)]
        fu16 = fc[pl.ds(u0, 16)]
        iu16 = ic[pl.ds(u0, 16)]
        ee_d = ec[pl.ds(u0, 16)]
        for g in range(4):
            ku = [ku16[4 * g + j] for j in range(4)]
            fu = [fu16[4 * g + j] for j in range(4)]
            iu = [iu16[4 * g + j] for j in range(4)]

            @plsc.parallel_loop(0, ub, unroll=4, carry=(zero16,) * 4)
            def v_lt(cj, a):
                v0 = cj * 16
                kk = kc[pl.ds(v0, 16)]
                ee = ec[pl.ds(v0, 16)]
                return tuple(a[j] + jnp.where(kk < ku[j], ee, 0.0)
                             for j in range(4))

            @plsc.parallel_loop(ub + 1, ncv, unroll=4, carry=v_lt)
            def v_le(cj, a):
                v0 = cj * 16
                kk = kc[pl.ds(v0, 16)]
                ee = ec[pl.ds(v0, 16)]
                return tuple(a[j] + jnp.where(kk <= ku[j], ee, 0.0)
                             for j in range(4))

            a4 = v_le
            for j in range(4):
                cond = (ku16 < ku[j]) | ((ku16 == ku[j]) & (iu16 > iu[j]))
                av = a4[j] + jnp.where(cond, ee_d, 0.0)
                acc = acc + fu[j] * av
        return acc

    acc = lax.fori_loop(b0, b1, u_loop, zero16)
    accv[...] = acc
    pcov[...] = jnp.broadcast_to(jnp.float32(pc), (16,))
    pltpu.sync_copy(accv, out_hbm.at[wid])
    pltpu.sync_copy(pcov, pout_hbm.at[wid])


@jax.jit
def kernel(cls, label_cls, pred_bboxes, label_target):
    logit = cls.reshape(B, N, 2)[:, :, 1]
    logit = jnp.pad(logit, ((0, 0), (0, NP - N)))
    lab = jnp.pad(label_cls.reshape(B, N).astype(jnp.float32),
                  ((0, 0), (0, NP - N)))
    bbox = jnp.pad(pred_bboxes, ((0, 0), (0, 0), (0, NP - N)))
    tgt = jnp.pad(label_target, ((0, 0), (0, 12)))

    sc_call = functools.partial(
        pl.kernel,
        out_type=[jax.ShapeDtypeStruct((2 * TASKS, 16), jnp.float32),
                  jax.ShapeDtypeStruct((2 * TASKS, 16), jnp.float32)],
        mesh=plsc.VectorSubcoreMesh(core_axis_name="c", subcore_axis_name="s"),
        compiler_params=pltpu.CompilerParams(needs_layout_passes=False),
        scratch_types=[
            pltpu.VMEM((NP,), jnp.float32),
            pltpu.VMEM((NP,), jnp.float32),
            pltpu.VMEM((NP,), jnp.float32),
            pltpu.VMEM((NP,), jnp.float32),
            pltpu.VMEM((NP,), jnp.float32),
            pltpu.VMEM((NP,), jnp.float32),
            pltpu.VMEM((16,), jnp.float32),
            pltpu.VMEM((NP + 16,), jnp.float32),
            pltpu.VMEM((NP + 16,), jnp.float32),
            pltpu.VMEM((NP + 16,), jnp.float32),
            pltpu.VMEM((NP + 16,), jnp.int32),
            pltpu.VMEM((16,), jnp.float32),
            pltpu.VMEM((16,), jnp.float32),
        ],
    )(_sc_body)
    partials, pcout = sc_call(logit, lab, bbox, tgt)

    sums = jnp.sum(partials.reshape(B, 2, 2 * 16), axis=2)
    p = pcout[::4, 0]
    cnt = p * (p - 1.0) * 0.5
    loss1 = sums[:, 0] / cnt
    loss2 = sums[:, 1] / cnt
    valid = (p > 1.0) & ~jnp.isnan(loss1) & ~jnp.isnan(loss2)
    l1 = jnp.where(valid, loss1, 0.0)
    l2 = jnp.where(valid, loss2, 0.0)
    nvalid = jnp.sum(valid.astype(jnp.float32))
    final1 = jnp.where(nvalid > 0, jnp.sum(l1) / nvalid, 0.0)
    final2 = jnp.where(nvalid > 0, jnp.sum(l2) / nvalid, 0.0)
    return (final1, final2)


# speedup vs baseline: 11.7114x; 1.0466x over previous
import functools

import jax
import jax.numpy as jnp
from jax import lax
from jax.experimental import pallas as pl
from jax.experimental.pallas import tpu as pltpu
from jax.experimental.pallas import tpu_sc as plsc

N = 3125
NP = 3328
B = 8
TASKS = 2 * B
NCH = NP // 16


def _sc_body(logit_hbm, lab_hbm, bbox_hbm, tgt_hbm, out_hbm, pout_hbm,
             labv, probv, x1v, y1v, x2v, y2v, tgtv,
             kc, ec, fc, ic, accv, pcov):
    c = lax.axis_index("c")
    s = lax.axis_index("s")
    wid = s * 2 + c
    task = wid // 2
    half = wid % 2
    b = task // 2
    l0 = (task % 2) == 0

    pltpu.sync_copy(lab_hbm.at[b], labv)
    pltpu.sync_copy(logit_hbm.at[b], probv)
    pltpu.sync_copy(bbox_hbm.at[b, 0], x1v)
    pltpu.sync_copy(bbox_hbm.at[b, 1], y1v)
    pltpu.sync_copy(bbox_hbm.at[b, 2], x2v)
    pltpu.sync_copy(bbox_hbm.at[b, 3], y2v)
    pltpu.sync_copy(tgt_hbm.at[b], tgtv)
    t16 = tgtv[...]
    tx1 = t16[0]
    ty1 = t16[1]
    tx2 = t16[2]
    ty2 = t16[3]
    ta = (tx2 - tx1) * (ty2 - ty1)

    iota = lax.iota(jnp.int32, 16)
    zero16 = jnp.zeros((16,), jnp.float32)

    def prob_loop(cj, mm):
        v0 = cj * 16
        pos = labv[pl.ds(v0, 16)] > 0.0
        prob = jnp.exp(probv[pl.ds(v0, 16)])
        probv[pl.ds(v0, 16)] = prob
        mn = jnp.minimum(mm[0], jnp.where(pos, prob, jnp.inf))
        mx = jnp.maximum(mm[1], jnp.where(pos, prob, -jnp.inf))
        return (mn, mx)

    mn16, mx16 = lax.fori_loop(0, NCH, prob_loop,
                               (jnp.full((16,), jnp.inf, jnp.float32),
                                jnp.full((16,), -jnp.inf, jnp.float32)))
    s1 = 0.5 * (jnp.min(mn16) + jnp.max(mx16))
    sh = jnp.where(l0, s1, 0.5)

    def comp_loop(cj, cnt):
        v0 = cj * 16
        pos = labv[pl.ds(v0, 16)] > 0.0
        x1 = x1v[pl.ds(v0, 16)]
        y1 = y1v[pl.ds(v0, 16)]
        x2 = x2v[pl.ds(v0, 16)]
        y2 = y2v[pl.ds(v0, 16)]
        ww = jnp.maximum(jnp.minimum(tx2, x2) - jnp.maximum(tx1, x1), 0.0)
        hh = jnp.maximum(jnp.minimum(ty2, y2) - jnp.maximum(ty1, y1), 0.0)
        inter = ww * hh
        iou = inter / ((x2 - x1) * (y2 - y1) + ta - inter)
        prob = probv[pl.ds(v0, 16)]
        key = jnp.where(l0, iou, prob)
        val = jnp.where(l0, prob, iou)
        ee = jnp.where(pos, jnp.exp(val - sh), 0.0)
        ff = jnp.where(pos, jnp.exp(sh - val), 0.0)
        cs = jnp.where(pos, 1, 0)
        for k in (1, 2, 4, 8):
            g = cs.at[jnp.maximum(iota - k, 0)].get(mode="promise_in_bounds")
            cs = cs + jnp.where(iota >= k, g, 0)
        idx = cnt + cs - 1
        plsc.store_scatter(kc, [idx], key, mask=pos)
        plsc.store_scatter(ec, [idx], ee, mask=pos)
        plsc.store_scatter(fc, [idx], ff, mask=pos)
        plsc.store_scatter(ic, [idx], v0 + iota, mask=pos)
        return cnt + cs[15]

    pc = lax.fori_loop(0, NCH, comp_loop, jnp.int32(0))
    kc[pl.ds(pc, 16)] = zero16
    ec[pl.ds(pc, 16)] = zero16
    fc[pl.ds(pc, 16)] = zero16

    nb = (pc + 15) // 16
    ncv = nb
    b0 = jnp.where(half == 0, 0, nb // 2)
    b1 = jnp.where(half == 0, nb // 2, nb)

    def u_loop(ub, acc):
        u0 = ub * 16
        ku16 = kc[pl.ds(u0, 16)]
        fu16 = fc[pl.ds(u0, 16)]
        iu16 = ic[pl.ds(u0, 16)]
        ee_d = ec[pl.ds(u0, 16)]
        for g in range(2):
            ku = [ku16[8 * g + j] for j in range(8)]
            fu = [fu16[8 * g + j] for j in range(8)]
            iu = [iu16[8 * g + j] for j in range(8)]

            @plsc.parallel_loop(0, ub, unroll=2, carry=(zero16,) * 8)
            def v_lt(cj, a):
                v0 = cj * 16
                kk = kc[pl.ds(v0, 16)]
                ee = ec[pl.ds(v0, 16)]
                return tuple(a[j] + jnp.where(kk < ku[j], ee, 0.0)
                             for j in range(8))

            @plsc.parallel_loop(ub + 1, ncv, unroll=2, carry=v_lt)
            def v_le(cj, a):
                v0 = cj * 16
                kk = kc[pl.ds(v0, 16)]
                ee = ec[pl.ds(v0, 16)]
                return tuple(a[j] + jnp.where(kk <= ku[j], ee, 0.0)
                             for j in range(8))

            a8 = v_le
            for j in range(8):
                cond = (ku16 < ku[j]) | ((ku16 == ku[j]) & (iu16 > iu[j]))
                av = a8[j] + jnp.where(cond, ee_d, 0.0)
                acc = acc + fu[j] * av
        return acc

    acc = lax.fori_loop(b0, b1, u_loop, zero16)
    accv[...] = acc
    pcov[...] = jnp.broadcast_to(jnp.float32(pc), (16,))
    pltpu.sync_copy(accv, out_hbm.at[wid])
    pltpu.sync_copy(pcov, pout_hbm.at[wid])


@jax.jit
def kernel(cls, label_cls, pred_bboxes, label_target):
    logit = cls.reshape(B, N, 2)[:, :, 1]
    logit = jnp.pad(logit, ((0, 0), (0, NP - N)))
    lab = jnp.pad(label_cls.reshape(B, N).astype(jnp.float32),
                  ((0, 0), (0, NP - N)))
    bbox = jnp.pad(pred_bboxes, ((0, 0), (0, 0), (0, NP - N)))
    tgt = jnp.pad(label_target, ((0, 0), (0, 12)))

    sc_call = functools.partial(
        pl.kernel,
        out_type=[jax.ShapeDtypeStruct((2 * TASKS, 16), jnp.float32),
                  jax.ShapeDtypeStruct((2 * TASKS, 16), jnp.float32)],
        mesh=plsc.VectorSubcoreMesh(core_axis_name="c", subcore_axis_name="s"),
        compiler_params=pltpu.CompilerParams(needs_layout_passes=False),
        scratch_types=[
            pltpu.VMEM((NP,), jnp.float32),
            pltpu.VMEM((NP,), jnp.float32),
            pltpu.VMEM((NP,), jnp.float32),
            pltpu.VMEM((NP,), jnp.float32),
            pltpu.VMEM((NP,), jnp.float32),
            pltpu.VMEM((NP,), jnp.float32),
            pltpu.VMEM((16,), jnp.float32),
            pltpu.VMEM((NP + 16,), jnp.float32),
            pltpu.VMEM((NP + 16,), jnp.float32),
            pltpu.VMEM((NP + 16,), jnp.float32),
            pltpu.VMEM((NP + 16,), jnp.int32),
            pltpu.VMEM((16,), jnp.float32),
            pltpu.VMEM((16,), jnp.float32),
        ],
    )(_sc_body)
    partials, pcout = sc_call(logit, lab, bbox, tgt)

    sums = jnp.sum(partials.reshape(B, 2, 2 * 16), axis=2)
    p = pcout[::4, 0]
    cnt = p * (p - 1.0) * 0.5
    loss1 = sums[:, 0] / cnt
    loss2 = sums[:, 1] / cnt
    valid = (p > 1.0) & ~jnp.isnan(loss1) & ~jnp.isnan(loss2)
    l1 = jnp.where(valid, loss1, 0.0)
    l2 = jnp.where(valid, loss2, 0.0)
    nvalid = jnp.sum(valid.astype(jnp.float32))
    final1 = jnp.where(nvalid > 0, jnp.sum(l1) / nvalid, 0.0)
    final2 = jnp.where(nvalid > 0, jnp.sum(l2) / nvalid, 0.0)
    return (final1, final2)
